# baseline (speedup 1.0000x reference)
import functools

import jax
import jax.numpy as jnp
from jax.experimental import pallas as pl
from jax.experimental.pallas import tpu as pltpu

_B = 4
_N = 4096
_MT = 1024
_LANES = 128


def _chamfer_body(p1_ref, p2t_ref, out_ref, acc_ref, sum_ref):
    b = pl.program_id(0)
    j = pl.program_id(1)
    nj = pl.num_programs(1)

    p1 = p1_ref[0]
    p2t = p2t_ref[0]

    x2 = jnp.sum(p1 * p1, axis=1, keepdims=True)
    y2 = jnp.sum(p2t * p2t, axis=0, keepdims=True)
    xy = jax.lax.dot_general(
        p1, p2t, (((1,), (0,)), ((), ())),
        preferred_element_type=jnp.float32)
    d = jnp.maximum((x2 - 2.0 * xy) + y2, 0.0)

    s2 = jnp.sum(jnp.min(d, axis=0, keepdims=True))

    fold = d[:, 0:_LANES]
    for k in range(1, _MT // _LANES):
        fold = jnp.minimum(fold, d[:, k * _LANES:(k + 1) * _LANES])

    @pl.when(j == 0)
    def _():
        acc_ref[...] = fold

    @pl.when(j > 0)
    def _():
        acc_ref[...] = jnp.minimum(acc_ref[...], fold)

    @pl.when((b == 0) & (j == 0))
    def _():
        sum_ref[0] = 0.0

    sum_ref[0] += s2

    @pl.when(j == nj - 1)
    def _():
        sum_ref[0] += jnp.sum(jnp.min(acc_ref[...], axis=1))
        @pl.when(b == _B - 1)
        def _():
            out_ref[...] = jnp.broadcast_to(sum_ref[0], (1, 1))


@jax.jit
def kernel(points1, points2):
    p1 = jnp.pad(points1, ((0, 0), (0, 0), (0, 5)))
    p2t = jnp.pad(points2.transpose(0, 2, 1), ((0, 0), (0, 5), (0, 0)))

    out = pl.pallas_call(
        _chamfer_body,
        grid=(_B, _N // _MT),
        in_specs=[
            pl.BlockSpec((1, _N, 8), lambda b, j: (b, 0, 0)),
            pl.BlockSpec((1, 8, _MT), lambda b, j: (b, 0, j)),
        ],
        out_specs=pl.BlockSpec((1, 1), lambda b, j: (0, 0)),
        out_shape=jax.ShapeDtypeStruct((1, 1), jnp.float32),
        scratch_shapes=[
            pltpu.VMEM((_N, _LANES), jnp.float32),
            pltpu.SMEM((1,), jnp.float32),
        ],
    )(p1, p2t)
    return out[0, 0]


# speedup vs baseline: 1.6026x; 1.6026x over previous
import jax
import jax.numpy as jnp
from jax.experimental import pallas as pl
from jax.experimental.pallas import tpu as pltpu

_B = 4
_N = 4096
_MT = 2048
_LANES = 128


def _chamfer_body(a_ref, bt_ref, out_ref, acc_ref, sum_ref):
    b = pl.program_id(0)
    j = pl.program_id(1)
    nj = pl.num_programs(1)

    a = a_ref[0]
    bt = bt_ref[0]

    d = jax.lax.dot_general(
        a, bt, (((1,), (0,)), ((), ())),
        preferred_element_type=jnp.float32)

    s2 = jnp.sum(jnp.maximum(jnp.min(d, axis=0, keepdims=True), 0.0))

    fold = d[:, 0:_LANES]
    for k in range(1, _MT // _LANES):
        fold = jnp.minimum(fold, d[:, k * _LANES:(k + 1) * _LANES])

    @pl.when(j == 0)
    def _():
        acc_ref[...] = fold

    @pl.when(j > 0)
    def _():
        acc_ref[...] = jnp.minimum(acc_ref[...], fold)

    @pl.when((b == 0) & (j == 0))
    def _():
        sum_ref[0] = 0.0

    sum_ref[0] += s2

    @pl.when(j == nj - 1)
    def _():
        sum_ref[0] += jnp.sum(
            jnp.maximum(jnp.min(acc_ref[...], axis=1), 0.0))
        @pl.when(b == _B - 1)
        def _():
            out_ref[...] = jnp.broadcast_to(sum_ref[0], (1, 1))


@jax.jit
def kernel(points1, points2):
    x2 = jnp.sum(points1 * points1, axis=-1, keepdims=True)
    y2 = jnp.sum(points2 * points2, axis=-1, keepdims=True)
    ones = jnp.ones_like(x2)
    zeros = jnp.zeros((_B, _N, 3), jnp.float32)
    a = jnp.concatenate([-2.0 * points1, x2, ones, zeros], axis=-1)
    bm = jnp.concatenate([points2, ones, y2, zeros], axis=-1)
    bt = bm.transpose(0, 2, 1)

    out = pl.pallas_call(
        _chamfer_body,
        grid=(_B, _N // _MT),
        in_specs=[
            pl.BlockSpec((1, _N, 8), lambda b, j: (b, 0, 0)),
            pl.BlockSpec((1, 8, _MT), lambda b, j: (b, 0, j)),
        ],
        out_specs=pl.BlockSpec((1, 1), lambda b, j: (0, 0)),
        out_shape=jax.ShapeDtypeStruct((1, 1), jnp.float32),
        scratch_shapes=[
            pltpu.VMEM((_N, _LANES), jnp.float32),
            pltpu.SMEM((1,), jnp.float32),
        ],
    )(a, bt)
    return out[0, 0]
